# baseline (speedup 1.0000x reference)
import functools

import jax
import jax.numpy as jnp
from jax import lax
from jax.experimental import pallas as pl
from jax.experimental.pallas import tpu as pltpu
from jax.experimental.pallas import tpu_sc as plsc

_TOPK = 2
_NC = 1
_NS = 16
_L = 16
_NW = _NC * _NS


def _matmul_body(x_ref, w_ref, b_ref, gl_ref):
    gl_ref[...] = jnp.dot(x_ref[...], w_ref[...],
                          preferred_element_type=jnp.float32) + b_ref[...]


def _matmul_tc(x, wt, b2, blk, rows, blk_off):
    hidden = x.shape[1]
    experts = wt.shape[1]
    return pl.pallas_call(
        _matmul_body,
        grid=(rows // blk,),
        in_specs=[
            pl.BlockSpec((blk, hidden), lambda i: (i + blk_off, 0)),
            pl.BlockSpec((hidden, experts), lambda i: (0, 0)),
            pl.BlockSpec((1, experts), lambda i: (0, 0)),
        ],
        out_specs=pl.BlockSpec((blk, experts), lambda i: (i, 0)),
        out_shape=jax.ShapeDtypeStruct((rows, experts), jnp.float32),
    )(x, wt, b2)


def _topk_sc(logits, tokens, experts):
    rows_w = tokens // _NW
    flat_w = rows_w * experts
    groups = rows_w // _L
    unroll = 1
    mesh = plsc.VectorSubcoreMesh(core_axis_name="c", subcore_axis_name="s",
                                  num_cores=_NC)

    @functools.partial(
        pl.kernel,
        out_type=(
            jax.ShapeDtypeStruct((tokens * experts,), jnp.float32),
            jax.ShapeDtypeStruct((tokens * _TOPK,), jnp.int32),
        ),
        mesh=mesh,
        scratch_types=[
            pltpu.VMEM((flat_w,), jnp.float32),
            pltpu.VMEM((flat_w,), jnp.float32),
            pltpu.VMEM((rows_w * _TOPK,), jnp.int32),
        ],
        compiler_params=pltpu.CompilerParams(use_tc_tiling_on_sc=False,
                                             needs_layout_passes=False),
    )
    def sc_kernel(lg_hbm, sp_hbm, idx_hbm, lg_v, sp_v, idx_v):
        w = lax.axis_index("s") * _NC + lax.axis_index("c")
        fbase = w * flat_w
        pltpu.sync_copy(lg_hbm.at[pl.ds(fbase, flat_w)], lg_v)
        neg = jnp.full((_L,), -jnp.inf, dtype=jnp.float32)
        lanes = lax.iota(jnp.int32, _L)

        def group(gq, carry):
            g0 = gq * unroll
            for j in range(unroll * experts):
                sp_v[pl.ds(g0 * (_L * experts) + j * _L, _L)] = neg
            fb = [(g0 + u) * _L * experts + lanes * experts
                  for u in range(unroll)]
            m1 = [neg] * unroll
            m2 = [neg] * unroll
            i1 = [jnp.zeros((_L,), jnp.int32)] * unroll
            i2 = [jnp.zeros((_L,), jnp.int32)] * unroll
            for e in range(experts):
                ev = jnp.full((_L,), e, dtype=jnp.int32)
                for u in range(unroll):
                    v = plsc.load_gather(lg_v, [fb[u] + e])
                    gt1 = v > m1[u]
                    gt2 = v > m2[u]
                    i2[u] = jnp.where(gt1, i1[u], jnp.where(gt2, ev, i2[u]))
                    m2[u] = jnp.where(gt1, m1[u], jnp.where(gt2, v, m2[u]))
                    i1[u] = jnp.where(gt1, ev, i1[u])
                    m1[u] = jnp.where(gt1, v, m1[u])
            for u in range(unroll):
                rowidx = (g0 + u) * _L + lanes
                plsc.store_scatter(sp_v, [fb[u] + i1[u]], m1[u])
                plsc.store_scatter(sp_v, [fb[u] + i2[u]], m2[u])
                plsc.store_scatter(idx_v, [rowidx * _TOPK], i1[u])
                plsc.store_scatter(idx_v, [rowidx * _TOPK + 1], i2[u])
            return carry

        lax.fori_loop(0, groups // unroll, group, 0)
        pltpu.sync_copy(sp_v, sp_hbm.at[pl.ds(fbase, flat_w)])
        ibase = w * rows_w * _TOPK
        pltpu.sync_copy(idx_v, idx_hbm.at[pl.ds(ibase, rows_w * _TOPK)])

    return sc_kernel(logits.reshape(tokens * experts))


@jax.jit
def kernel(x, W, b):
    tokens, hidden = x.shape
    experts = W.shape[0]
    wt = W.T
    b2 = b.reshape(1, experts)
    logits = _matmul_tc(x, wt, b2, 2048, tokens, 0)
    sp_flat, idx_flat = _topk_sc(logits, tokens, experts)
    return (sp_flat.reshape(tokens, experts),
            idx_flat.reshape(tokens, _TOPK),
            logits)


# speedup vs baseline: 1.2331x; 1.2331x over previous
import functools

import jax
import jax.numpy as jnp
from jax import lax
from jax.experimental import pallas as pl
from jax.experimental.pallas import tpu as pltpu
from jax.experimental.pallas import tpu_sc as plsc

_TOPK = 2
_NC = 2
_NS = 16
_L = 16
_NW = _NC * _NS


def _matmul_body(x_ref, w_ref, b_ref, gl_ref, glt_ref):
    logits = jnp.dot(x_ref[...], w_ref[...],
                     preferred_element_type=jnp.float32) + b_ref[...]
    gl_ref[...] = logits
    glt_ref[...] = logits.T


def _matmul_tc(x, wt, b2, blk, rows, blk_off):
    hidden = x.shape[1]
    experts = wt.shape[1]
    return pl.pallas_call(
        _matmul_body,
        grid=(rows // blk,),
        in_specs=[
            pl.BlockSpec((blk, hidden), lambda i: (i + blk_off, 0)),
            pl.BlockSpec((hidden, experts), lambda i: (0, 0)),
            pl.BlockSpec((1, experts), lambda i: (0, 0)),
        ],
        out_specs=(
            pl.BlockSpec((blk, experts), lambda i: (i, 0)),
            pl.BlockSpec((experts, blk), lambda i: (0, i)),
        ),
        out_shape=(
            jax.ShapeDtypeStruct((rows, experts), jnp.float32),
            jax.ShapeDtypeStruct((experts, rows), jnp.float32),
        ),
    )(x, wt, b2)


def _topk_sc(logits_t, tokens, experts):
    rows_w = tokens // _NW
    flat_w = rows_w * experts
    groups = rows_w // _L
    mesh = plsc.VectorSubcoreMesh(core_axis_name="c", subcore_axis_name="s",
                                  num_cores=_NC)

    @functools.partial(
        pl.kernel,
        out_type=(
            jax.ShapeDtypeStruct((tokens * experts,), jnp.float32),
            jax.ShapeDtypeStruct((tokens * _TOPK,), jnp.int32),
        ),
        mesh=mesh,
        scratch_types=[
            pltpu.VMEM((experts, rows_w), jnp.float32),
            pltpu.VMEM((flat_w,), jnp.float32),
            pltpu.VMEM((rows_w * _TOPK,), jnp.int32),
        ],
        compiler_params=pltpu.CompilerParams(use_tc_tiling_on_sc=False,
                                             needs_layout_passes=False),
    )
    def sc_kernel(lgt_hbm, sp_hbm, idx_hbm, lgt_v, sp_v, idx_v):
        w = lax.axis_index("s") * _NC + lax.axis_index("c")
        rbase = w * rows_w
        pltpu.sync_copy(lgt_hbm.at[:, pl.ds(rbase, rows_w)], lgt_v)
        neg = jnp.full((_L,), -jnp.inf, dtype=jnp.float32)
        lanes = lax.iota(jnp.int32, _L)

        def group(g, carry):
            gb = g * (_L * experts)
            for j in range(experts):
                sp_v[pl.ds(gb + j * _L, _L)] = neg
            flatbase = gb + lanes * experts
            m1 = neg
            m2 = neg
            i1 = jnp.zeros((_L,), jnp.int32)
            i2 = i1
            for e in range(experts):
                v = lgt_v[e, pl.ds(g * _L, _L)]
                gt1 = v > m1
                gt2 = v > m2
                ev = jnp.full((_L,), e, dtype=jnp.int32)
                i2 = jnp.where(gt1, i1, jnp.where(gt2, ev, i2))
                m2 = jnp.where(gt1, m1, jnp.where(gt2, v, m2))
                i1 = jnp.where(gt1, ev, i1)
                m1 = jnp.where(gt1, v, m1)
            plsc.store_scatter(sp_v, [flatbase + i1], m1)
            plsc.store_scatter(sp_v, [flatbase + i2], m2)
            rowidx = g * _L + lanes
            plsc.store_scatter(idx_v, [rowidx * _TOPK], i1)
            plsc.store_scatter(idx_v, [rowidx * _TOPK + 1], i2)
            return carry

        lax.fori_loop(0, groups, group, 0)
        fbase = w * flat_w
        pltpu.sync_copy(sp_v, sp_hbm.at[pl.ds(fbase, flat_w)])
        ibase = w * rows_w * _TOPK
        pltpu.sync_copy(idx_v, idx_hbm.at[pl.ds(ibase, rows_w * _TOPK)])

    return sc_kernel(logits_t)


@jax.jit
def kernel(x, W, b):
    tokens, hidden = x.shape
    experts = W.shape[0]
    wt = W.T
    b2 = b.reshape(1, experts)
    logits, logits_t = _matmul_tc(x, wt, b2, 2048, tokens, 0)
    sp_flat, idx_flat = _topk_sc(logits_t, tokens, experts)
    return (sp_flat.reshape(tokens, experts),
            idx_flat.reshape(tokens, _TOPK),
            logits)
